# baseline (speedup 1.0000x reference)
import functools

import jax
import jax.numpy as jnp
from jax.experimental import pallas as pl
from jax.experimental.pallas import tpu as pltpu

_B, _N, _IN = 16, 32768, 32
_HID = 64
_KSEL = 8
_LAM = 0.5
_RSEL = 1.0
_KEFF = 16
_TN = 4096
_LANES = 128
_ROWS = _N // _LANES


def _saliency_kernel(x_ref, w1_ref, b1_ref, w2_ref, b2_ref, out_ref):
    xb = x_ref[0]
    h = jnp.tanh(
        jnp.dot(xb, w1_ref[...], preferred_element_type=jnp.float32)
        + b1_ref[...]
    )
    es = jnp.dot(h, w2_ref[...], preferred_element_type=jnp.float32)
    es = es + b2_ref[0, 0]
    sal = jax.nn.softplus(es)
    out_ref[0] = sal.reshape(_TN // _LANES, _LANES)


def _select_kernel(sal_ref, pos_ref, x_hbm, wlx_ref, wls_ref, wlp_ref,
                   wlc_ref, bl_ref, wp_ref, bp_ref,
                   y_ref, tok_ref,
                   row_vmem, idx_smem, sem):
    b = pl.program_id(0)
    s = sal_ref[0]
    t = s * (_RSEL / _LAM)
    m = jnp.max(t)
    e = jnp.exp(t - m)
    z = jnp.sum(e)
    y = e * (_KSEL / z)
    y_ref[0] = y

    r_iota = jax.lax.broadcasted_iota(jnp.int32, (_ROWS, _LANES), 0)
    c_iota = jax.lax.broadcasted_iota(jnp.int32, (_ROWS, _LANES), 1)
    flat = r_iota * _LANES + c_iota

    yw = y
    for k in range(_KEFF):
        v = jnp.max(yw)
        idxk = jnp.min(jnp.where(yw == v, flat, jnp.int32(_N)))
        idx_smem[k] = idxk
        salk = jnp.sum(jnp.where(flat == idxk, s, 0.0))
        cumk = jnp.sum(jnp.where(flat <= idxk, s, 0.0)) * (1.0 / _N)
        posk = jnp.sum(jnp.where(flat == idxk, pos_ref[...], 0.0))
        yw = jnp.where(flat == idxk, -1.0, yw)

        idx_s = idx_smem[k]
        cp = pltpu.make_async_copy(
            x_hbm.at[b, pl.ds(idx_s, 1), :],
            row_vmem.at[pl.ds(k, 1), :],
            sem,
        )
        cp.start()
        cp.wait()
        xrow = row_vmem[pl.ds(k, 1), :]

        n2 = (jnp.sum(xrow * xrow) + salk * salk + posk * posk + cumk * cumk)
        denom = jnp.sqrt(n2) + 1e-6
        g = (
            jnp.dot(xrow, wlx_ref[...], preferred_element_type=jnp.float32)
            + salk * wls_ref[...]
            + posk * wlp_ref[...]
            + cumk * wlc_ref[...]
        )
        lifted = jnp.tanh(g / denom + bl_ref[...])
        tok = (
            jnp.dot(lifted, wp_ref[...], preferred_element_type=jnp.float32)
            + bp_ref[...]
        )
        tok_ref[0, pl.ds(k, 1), :] = tok


@functools.partial(jax.jit, static_argnums=())
def kernel(x, W1, b1, W2, b2, W_lift, b_lift, Wp, bp):
    B, N, IN = x.shape
    d_model = Wp.shape[1]

    sal = pl.pallas_call(
        _saliency_kernel,
        grid=(B, N // _TN),
        in_specs=[
            pl.BlockSpec((1, _TN, IN), lambda b, t: (b, t, 0)),
            pl.BlockSpec((IN, _HID), lambda b, t: (0, 0)),
            pl.BlockSpec((1, _HID), lambda b, t: (0, 0)),
            pl.BlockSpec((_HID, 1), lambda b, t: (0, 0)),
            pl.BlockSpec((1, 1), lambda b, t: (0, 0)),
        ],
        out_specs=pl.BlockSpec(
            (1, _TN // _LANES, _LANES), lambda b, t: (b, t, 0)
        ),
        out_shape=jax.ShapeDtypeStruct((B, _ROWS, _LANES), jnp.float32),
    )(x, W1, b1.reshape(1, _HID), W2, b2.reshape(1, 1))

    pos_arr = jnp.linspace(0.0, 1.0, N, dtype=jnp.float32).reshape(
        _ROWS, _LANES
    )

    y_star, tokens = pl.pallas_call(
        _select_kernel,
        grid=(B,),
        in_specs=[
            pl.BlockSpec((1, _ROWS, _LANES), lambda b: (b, 0, 0)),
            pl.BlockSpec((_ROWS, _LANES), lambda b: (0, 0)),
            pl.BlockSpec(memory_space=pl.ANY),
            pl.BlockSpec((IN, 16), lambda b: (0, 0)),
            pl.BlockSpec((1, 16), lambda b: (0, 0)),
            pl.BlockSpec((1, 16), lambda b: (0, 0)),
            pl.BlockSpec((1, 16), lambda b: (0, 0)),
            pl.BlockSpec((1, 16), lambda b: (0, 0)),
            pl.BlockSpec((16, d_model), lambda b: (0, 0)),
            pl.BlockSpec((1, d_model), lambda b: (0, 0)),
        ],
        out_specs=[
            pl.BlockSpec((1, _ROWS, _LANES), lambda b: (b, 0, 0)),
            pl.BlockSpec((1, _KEFF, d_model), lambda b: (b, 0, 0)),
        ],
        out_shape=[
            jax.ShapeDtypeStruct((B, _ROWS, _LANES), jnp.float32),
            jax.ShapeDtypeStruct((B, _KEFF, d_model), jnp.float32),
        ],
        scratch_shapes=[
            pltpu.VMEM((_KEFF, IN), jnp.float32),
            pltpu.SMEM((_KEFF,), jnp.int32),
            pltpu.SemaphoreType.DMA,
        ],
    )(
        sal,
        pos_arr,
        x,
        W_lift[:IN],
        W_lift[IN : IN + 1],
        W_lift[IN + 1 : IN + 2],
        W_lift[IN + 2 : IN + 3],
        b_lift.reshape(1, 16),
        Wp,
        bp.reshape(1, d_model),
    )

    return tokens, y_star.reshape(B, N)


# speedup vs baseline: 1.5440x; 1.5440x over previous
import jax
import jax.numpy as jnp
from jax.experimental import pallas as pl
from jax.experimental.pallas import tpu as pltpu

_B, _N, _IN = 16, 32768, 32
_HID = 64
_KSEL = 8
_LAM = 0.5
_RSEL = 1.0
_KEFF = 16
_TN = 4096
_LANES = 128
_ROWS = _N // _LANES


def _saliency_kernel(x_ref, w1t_ref, b1_ref, w2_ref, b2_ref, out_ref):
    xt = x_ref[0].T
    h = jnp.tanh(
        jnp.dot(w1t_ref[...], xt, preferred_element_type=jnp.float32)
        + b1_ref[...]
    )
    es = jnp.dot(w2_ref[...], h, preferred_element_type=jnp.float32)
    es = es + b2_ref[0, 0]
    sal = jax.nn.softplus(es)
    out_ref[0] = sal.reshape(_TN // _LANES, _LANES)


def _select_kernel(sal_ref, x_hbm, wlx_ref, wls_ref, wlp_ref,
                   wlc_ref, bl_ref, wp_ref, bp_ref,
                   y_ref, tok_ref,
                   rows_vmem, idx_smem, sems):
    b = pl.program_id(0)
    s = sal_ref[0]
    t = s * (_RSEL / _LAM)
    m = jnp.max(t)
    e = jnp.exp(t - m)
    z = jnp.sum(e)
    c8 = _KSEL / z
    y = e * c8
    y_ref[0] = y
    lc8 = jnp.log(c8)

    r_iota = jax.lax.broadcasted_iota(jnp.int32, (_ROWS, _LANES), 0)
    c_iota = jax.lax.broadcasted_iota(jnp.int32, (_ROWS, _LANES), 1)
    flat = r_iota * _LANES + c_iota
    k_iota = jax.lax.broadcasted_iota(jnp.int32, (_KEFF, 16), 0)

    yw = y
    salmat = jnp.zeros((_KEFF, 16), jnp.float32)
    posmat = jnp.zeros((_KEFF, 16), jnp.float32)
    cummat = jnp.zeros((_KEFF, 16), jnp.float32)
    copies = []
    for k in range(_KEFF):
        v = jnp.max(yw)
        idxk = jnp.min(jnp.where(yw == v, flat, jnp.int32(_N)))
        idx_smem[k] = idxk
        cp = pltpu.make_async_copy(
            x_hbm.at[b, pl.ds(idx_smem[k], 1), :],
            rows_vmem.at[pl.ds(k, 1), :],
            sems.at[k],
        )
        cp.start()
        copies.append(cp)

        salk = 0.5 * (jnp.log(v) - lc8 + m)
        posk = idxk.astype(jnp.float32) * (1.0 / (_N - 1))
        cumk = jnp.sum(jnp.where(flat <= idxk, s, 0.0)) * (1.0 / _N)
        km = k_iota == k
        salmat = jnp.where(km, salk, salmat)
        posmat = jnp.where(km, posk, posmat)
        cummat = jnp.where(km, cumk, cummat)
        yw = jnp.where(flat == idxk, -1.0, yw)

    for cp in copies:
        cp.wait()

    rows = rows_vmem[...]
    rowsq = jnp.sum(rows * rows, axis=1, keepdims=True)
    n2 = rowsq + (salmat * salmat + posmat * posmat + cummat * cummat)[:, 0:1]
    denom = jnp.sqrt(n2) + 1e-6
    g = (
        jnp.dot(rows, wlx_ref[...], preferred_element_type=jnp.float32)
        + salmat * wls_ref[...]
        + posmat * wlp_ref[...]
        + cummat * wlc_ref[...]
    )
    lifted = jnp.tanh(g / denom + bl_ref[...])
    tok = (
        jnp.dot(lifted, wp_ref[...], preferred_element_type=jnp.float32)
        + bp_ref[...]
    )
    tok_ref[0] = tok


def kernel(x, W1, b1, W2, b2, W_lift, b_lift, Wp, bp):
    B, N, IN = x.shape
    d_model = Wp.shape[1]

    sal = pl.pallas_call(
        _saliency_kernel,
        grid=(B, N // _TN),
        in_specs=[
            pl.BlockSpec((1, _TN, IN), lambda b, t: (b, t, 0)),
            pl.BlockSpec((_HID, IN), lambda b, t: (0, 0)),
            pl.BlockSpec((_HID, 1), lambda b, t: (0, 0)),
            pl.BlockSpec((1, _HID), lambda b, t: (0, 0)),
            pl.BlockSpec((1, 1), lambda b, t: (0, 0)),
        ],
        out_specs=pl.BlockSpec(
            (1, _TN // _LANES, _LANES), lambda b, t: (b, t, 0)
        ),
        out_shape=jax.ShapeDtypeStruct((B, _ROWS, _LANES), jnp.float32),
    )(x, W1.T, b1.reshape(_HID, 1), W2.T, b2.reshape(1, 1))

    y_star, tokens = pl.pallas_call(
        _select_kernel,
        grid=(B,),
        in_specs=[
            pl.BlockSpec((1, _ROWS, _LANES), lambda b: (b, 0, 0)),
            pl.BlockSpec(memory_space=pl.ANY),
            pl.BlockSpec((IN, 16), lambda b: (0, 0)),
            pl.BlockSpec((1, 16), lambda b: (0, 0)),
            pl.BlockSpec((1, 16), lambda b: (0, 0)),
            pl.BlockSpec((1, 16), lambda b: (0, 0)),
            pl.BlockSpec((1, 16), lambda b: (0, 0)),
            pl.BlockSpec((16, d_model), lambda b: (0, 0)),
            pl.BlockSpec((1, d_model), lambda b: (0, 0)),
        ],
        out_specs=[
            pl.BlockSpec((1, _ROWS, _LANES), lambda b: (b, 0, 0)),
            pl.BlockSpec((1, _KEFF, d_model), lambda b: (b, 0, 0)),
        ],
        out_shape=[
            jax.ShapeDtypeStruct((B, _ROWS, _LANES), jnp.float32),
            jax.ShapeDtypeStruct((B, _KEFF, d_model), jnp.float32),
        ],
        scratch_shapes=[
            pltpu.VMEM((_KEFF, IN), jnp.float32),
            pltpu.SMEM((_KEFF,), jnp.int32),
            pltpu.SemaphoreType.DMA((_KEFF,)),
        ],
    )(
        sal,
        x,
        W_lift[:IN],
        W_lift[IN : IN + 1],
        W_lift[IN + 1 : IN + 2],
        W_lift[IN + 2 : IN + 3],
        b_lift.reshape(1, 16),
        Wp,
        bp.reshape(1, d_model),
    )

    return tokens, y_star.reshape(B, N)


# speedup vs baseline: 3.9508x; 2.5589x over previous
import jax
import jax.numpy as jnp
from jax.experimental import pallas as pl
from jax.experimental.pallas import tpu as pltpu

_B, _N, _IN = 16, 32768, 32
_HID = 64
_KSEL = 8
_LAM = 0.5
_RSEL = 1.0
_KEFF = 16
_CH = 8192
_LANES = 128
_ROWS = _N // _LANES


def _fused_kernel(xt_ref, x_any, w1t_ref, b1_ref, w2_ref, b2_ref,
                  wlxt_ref, wlst_ref, wlpt_ref, wlct_ref, blt_ref,
                  wp_ref, bp_ref,
                  y_ref, tok_ref,
                  rows_vmem, idx_smem, sems):
    b = pl.program_id(0)

    chunks = []
    for c0 in range(0, _N, _CH):
        xc = xt_ref[0, :, c0 : c0 + _CH]
        h = jnp.tanh(
            jnp.dot(w1t_ref[...], xc, preferred_element_type=jnp.float32)
            + b1_ref[...]
        )
        es = jnp.dot(w2_ref[...], h, preferred_element_type=jnp.float32)
        es = es + b2_ref[0, 0]
        chunks.append(jax.nn.softplus(es).reshape(_CH // _LANES, _LANES))
    s = jnp.concatenate(chunks, axis=0)

    t = s * (_RSEL / _LAM)
    m = jnp.max(t)
    e = jnp.exp(t - m)
    z = jnp.sum(e)
    c8 = _KSEL / z
    y = e * c8
    y_ref[0] = y
    lc8 = jnp.log(c8)

    r_iota = jax.lax.broadcasted_iota(jnp.int32, (_ROWS, _LANES), 0)
    c_iota = jax.lax.broadcasted_iota(jnp.int32, (_ROWS, _LANES), 1)
    flat = r_iota * _LANES + c_iota
    k16_iota = jax.lax.broadcasted_iota(jnp.int32, (1, _KEFF), 1)

    yw = y
    salrow = jnp.zeros((1, _KEFF), jnp.float32)
    posrow = jnp.zeros((1, _KEFF), jnp.float32)
    cumrow = jnp.zeros((1, _KEFF), jnp.float32)
    copies = []
    offs = []
    for k in range(_KEFF):
        v = jnp.max(yw)
        idxk = jnp.min(jnp.where(yw == v, flat, jnp.int32(_N)))
        idx_smem[k] = (idxk // _LANES) * _LANES
        offs.append(idxk - idx_smem[k])
        cp = pltpu.make_async_copy(
            x_any.at[b, :, pl.ds(pl.multiple_of(idx_smem[k], _LANES), _LANES)],
            rows_vmem.at[:, pl.ds(k * _LANES, _LANES)],
            sems.at[k],
        )
        cp.start()
        copies.append(cp)

        salk = 0.5 * (jnp.log(v) - lc8 + m)
        posk = idxk.astype(jnp.float32) * (1.0 / (_N - 1))
        cumk = jnp.sum(jnp.where(flat <= idxk, s, 0.0)) * (1.0 / _N)
        km = k16_iota == k
        salrow = jnp.where(km, salk, salrow)
        posrow = jnp.where(km, posk, posrow)
        cumrow = jnp.where(km, cumk, cumrow)
        yw = jnp.where(flat == idxk, -1.0, yw)

    for cp in copies:
        cp.wait()

    l_iota = jax.lax.broadcasted_iota(jnp.int32, (1, _LANES), 1)
    cols = []
    for k in range(_KEFF):
        tile = rows_vmem[:, k * _LANES : (k + 1) * _LANES]
        sel = jnp.where(l_iota == offs[k], tile, 0.0)
        cols.append(jnp.sum(sel, axis=1, keepdims=True))
    rows = jnp.concatenate(cols, axis=1)
    n2 = (
        jnp.sum(rows * rows, axis=0, keepdims=True)
        + salrow * salrow + posrow * posrow + cumrow * cumrow
    )
    denom = jnp.sqrt(n2) + 1e-6
    gt = (
        jnp.dot(wlxt_ref[...], rows, preferred_element_type=jnp.float32)
        + wlst_ref[...] * salrow
        + wlpt_ref[...] * posrow
        + wlct_ref[...] * cumrow
    )
    lifted_t = jnp.tanh(gt / denom + blt_ref[...])
    tok = jax.lax.dot_general(
        lifted_t, wp_ref[...],
        dimension_numbers=(((0,), (0,)), ((), ())),
        preferred_element_type=jnp.float32,
    )
    tok_ref[0] = tok + bp_ref[...]


def kernel(x, W1, b1, W2, b2, W_lift, b_lift, Wp, bp):
    B, N, IN = x.shape
    d_model = Wp.shape[1]

    xt = jnp.transpose(x, (0, 2, 1))

    y_star, tokens = pl.pallas_call(
        _fused_kernel,
        grid=(B,),
        in_specs=[
            pl.BlockSpec((1, IN, N), lambda b: (b, 0, 0)),
            pl.BlockSpec(memory_space=pl.ANY),
            pl.BlockSpec((_HID, IN), lambda b: (0, 0)),
            pl.BlockSpec((_HID, 1), lambda b: (0, 0)),
            pl.BlockSpec((1, _HID), lambda b: (0, 0)),
            pl.BlockSpec((1, 1), lambda b: (0, 0)),
            pl.BlockSpec((16, IN), lambda b: (0, 0)),
            pl.BlockSpec((16, 1), lambda b: (0, 0)),
            pl.BlockSpec((16, 1), lambda b: (0, 0)),
            pl.BlockSpec((16, 1), lambda b: (0, 0)),
            pl.BlockSpec((16, 1), lambda b: (0, 0)),
            pl.BlockSpec((16, d_model), lambda b: (0, 0)),
            pl.BlockSpec((1, d_model), lambda b: (0, 0)),
        ],
        out_specs=[
            pl.BlockSpec((1, _ROWS, _LANES), lambda b: (b, 0, 0)),
            pl.BlockSpec((1, _KEFF, d_model), lambda b: (b, 0, 0)),
        ],
        out_shape=[
            jax.ShapeDtypeStruct((B, _ROWS, _LANES), jnp.float32),
            jax.ShapeDtypeStruct((B, _KEFF, d_model), jnp.float32),
        ],
        scratch_shapes=[
            pltpu.VMEM((_IN, _KEFF * _LANES), jnp.float32),
            pltpu.SMEM((_KEFF,), jnp.int32),
            pltpu.SemaphoreType.DMA((_KEFF,)),
        ],
    )(
        xt,
        xt,
        W1.T,
        b1.reshape(_HID, 1),
        W2.T,
        b2.reshape(1, 1),
        W_lift[:IN].T,
        W_lift[IN : IN + 1].T,
        W_lift[IN + 1 : IN + 2].T,
        W_lift[IN + 2 : IN + 3].T,
        b_lift.reshape(16, 1),
        Wp,
        bp.reshape(1, d_model),
    )

    return tokens, y_star.reshape(B, N)


# speedup vs baseline: 4.2826x; 1.0840x over previous
import jax
import jax.numpy as jnp
from jax.experimental import pallas as pl
from jax.experimental.pallas import tpu as pltpu

_B, _N, _IN = 16, 32768, 32
_HID = 64
_KSEL = 8
_LAM = 0.5
_RSEL = 1.0
_KEFF = 16
_CH = 8192
_LANES = 128
_ROWS = _N // _LANES
_NB = 2


def _fused_kernel(xt_ref, x_any, w1t_ref, b1_ref, w2_ref, b2_ref,
                  wlxt_ref, wlst_ref, wlpt_ref, wlct_ref, blt_ref,
                  wp_ref, bp_ref,
                  y_ref, tok_ref,
                  rows_vmem, idx_smem, sems):
    i = pl.program_id(0)

    r_iota = jax.lax.broadcasted_iota(jnp.int32, (_ROWS, _LANES), 0)
    c_iota = jax.lax.broadcasted_iota(jnp.int32, (_ROWS, _LANES), 1)
    flat = r_iota * _LANES + c_iota
    k16_iota = jax.lax.broadcasted_iota(jnp.int32, (1, _KEFF), 1)
    l_iota = jax.lax.broadcasted_iota(jnp.int32, (1, _LANES), 1)

    s_l, y_l, m_l, lc8_l = [], [], [], []
    for bi in range(_NB):
        chunks = []
        for c0 in range(0, _N, _CH):
            xc = xt_ref[bi, :, c0 : c0 + _CH]
            h = jnp.tanh(
                jnp.dot(w1t_ref[...], xc, preferred_element_type=jnp.float32)
                + b1_ref[...]
            )
            es = jnp.dot(w2_ref[...], h, preferred_element_type=jnp.float32)
            es = es + b2_ref[0, 0]
            chunks.append(jax.nn.softplus(es).reshape(_CH // _LANES, _LANES))
        s = jnp.concatenate(chunks, axis=0)

        t = s * (_RSEL / _LAM)
        m = jnp.max(t)
        e = jnp.exp(t - m)
        z = jnp.sum(e)
        c8 = _KSEL / z
        y = e * c8
        y_ref[bi] = y
        s_l.append(s)
        y_l.append(y)
        m_l.append(m)
        lc8_l.append(jnp.log(c8))

    yw_l = list(y_l)
    salrow_l = [jnp.zeros((1, _KEFF), jnp.float32) for _ in range(_NB)]
    posrow_l = [jnp.zeros((1, _KEFF), jnp.float32) for _ in range(_NB)]
    cumrow_l = [jnp.zeros((1, _KEFF), jnp.float32) for _ in range(_NB)]
    offs = [[None] * _KEFF for _ in range(_NB)]
    copies = []
    for k in range(_KEFF):
        for bi in range(_NB):
            yw = yw_l[bi]
            v = jnp.max(yw)
            idxk = jnp.min(jnp.where(yw == v, flat, jnp.int32(_N)))
            sk = bi * _KEFF + k
            idx_smem[sk] = (idxk // _LANES) * _LANES
            offs[bi][k] = idxk - idx_smem[sk]
            cp = pltpu.make_async_copy(
                x_any.at[
                    i * _NB + bi, :,
                    pl.ds(pl.multiple_of(idx_smem[sk], _LANES), _LANES),
                ],
                rows_vmem.at[:, pl.ds(sk * _LANES, _LANES)],
                sems.at[sk],
            )
            cp.start()
            copies.append(cp)

            salk = 0.5 * (jnp.log(v) - lc8_l[bi] + m_l[bi])
            posk = idxk.astype(jnp.float32) * (1.0 / (_N - 1))
            cumk = jnp.sum(jnp.where(flat <= idxk, s_l[bi], 0.0)) * (1.0 / _N)
            km = k16_iota == k
            salrow_l[bi] = jnp.where(km, salk, salrow_l[bi])
            posrow_l[bi] = jnp.where(km, posk, posrow_l[bi])
            cumrow_l[bi] = jnp.where(km, cumk, cumrow_l[bi])
            yw_l[bi] = jnp.where(flat == idxk, -1.0, yw)

    for cp in copies:
        cp.wait()

    for bi in range(_NB):
        cols = []
        for k in range(_KEFF):
            sk = bi * _KEFF + k
            tile = rows_vmem[:, sk * _LANES : (sk + 1) * _LANES]
            sel = jnp.where(l_iota == offs[bi][k], tile, 0.0)
            cols.append(jnp.sum(sel, axis=1, keepdims=True))
        rows = jnp.concatenate(cols, axis=1)

        salrow, posrow, cumrow = salrow_l[bi], posrow_l[bi], cumrow_l[bi]
        n2 = (
            jnp.sum(rows * rows, axis=0, keepdims=True)
            + salrow * salrow + posrow * posrow + cumrow * cumrow
        )
        denom = jnp.sqrt(n2) + 1e-6
        gt = (
            jnp.dot(wlxt_ref[...], rows, preferred_element_type=jnp.float32)
            + wlst_ref[...] * salrow
            + wlpt_ref[...] * posrow
            + wlct_ref[...] * cumrow
        )
        lifted_t = jnp.tanh(gt / denom + blt_ref[...])
        tok = jax.lax.dot_general(
            lifted_t, wp_ref[...],
            dimension_numbers=(((0,), (0,)), ((), ())),
            preferred_element_type=jnp.float32,
        )
        tok_ref[bi] = tok + bp_ref[...]


def kernel(x, W1, b1, W2, b2, W_lift, b_lift, Wp, bp):
    B, N, IN = x.shape
    d_model = Wp.shape[1]

    xt = jnp.transpose(x, (0, 2, 1))

    y_star, tokens = pl.pallas_call(
        _fused_kernel,
        grid=(B // _NB,),
        in_specs=[
            pl.BlockSpec((_NB, IN, N), lambda i: (i, 0, 0)),
            pl.BlockSpec(memory_space=pl.ANY),
            pl.BlockSpec((_HID, IN), lambda i: (0, 0)),
            pl.BlockSpec((_HID, 1), lambda i: (0, 0)),
            pl.BlockSpec((1, _HID), lambda i: (0, 0)),
            pl.BlockSpec((1, 1), lambda i: (0, 0)),
            pl.BlockSpec((16, IN), lambda i: (0, 0)),
            pl.BlockSpec((16, 1), lambda i: (0, 0)),
            pl.BlockSpec((16, 1), lambda i: (0, 0)),
            pl.BlockSpec((16, 1), lambda i: (0, 0)),
            pl.BlockSpec((16, 1), lambda i: (0, 0)),
            pl.BlockSpec((16, d_model), lambda i: (0, 0)),
            pl.BlockSpec((1, d_model), lambda i: (0, 0)),
        ],
        out_specs=[
            pl.BlockSpec((_NB, _ROWS, _LANES), lambda i: (i, 0, 0)),
            pl.BlockSpec((_NB, _KEFF, d_model), lambda i: (i, 0, 0)),
        ],
        out_shape=[
            jax.ShapeDtypeStruct((B, _ROWS, _LANES), jnp.float32),
            jax.ShapeDtypeStruct((B, _KEFF, d_model), jnp.float32),
        ],
        scratch_shapes=[
            pltpu.VMEM((_IN, _NB * _KEFF * _LANES), jnp.float32),
            pltpu.SMEM((_NB * _KEFF,), jnp.int32),
            pltpu.SemaphoreType.DMA((_NB * _KEFF,)),
        ],
    )(
        xt,
        xt,
        W1.T,
        b1.reshape(_HID, 1),
        W2.T,
        b2.reshape(1, 1),
        W_lift[:IN].T,
        W_lift[IN : IN + 1].T,
        W_lift[IN + 1 : IN + 2].T,
        W_lift[IN + 2 : IN + 3].T,
        b_lift.reshape(16, 1),
        Wp,
        bp.reshape(1, d_model),
    )

    return tokens, y_star.reshape(B, N)


# speedup vs baseline: 4.3524x; 1.0163x over previous
import jax
import jax.numpy as jnp
from jax.experimental import pallas as pl
from jax.experimental.pallas import tpu as pltpu

_B, _N, _IN = 16, 32768, 32
_HID = 64
_KSEL = 8
_LAM = 0.5
_RSEL = 1.0
_KEFF = 16
_CH = 8192
_LANES = 128
_ROWS = _N // _LANES
_NB = 4


def _fused_kernel(xt_ref, x_any, w1t_ref, b1_ref, w2_ref, b2_ref,
                  wlxt_ref, wlst_ref, wlpt_ref, wlct_ref, blt_ref,
                  wp_ref, bp_ref,
                  y_ref, tok_ref,
                  rows_vmem, idx_smem, sems):
    i = pl.program_id(0)

    r_iota = jax.lax.broadcasted_iota(jnp.int32, (_ROWS, _LANES), 0)
    c_iota = jax.lax.broadcasted_iota(jnp.int32, (_ROWS, _LANES), 1)
    flat = r_iota * _LANES + c_iota
    k16_iota = jax.lax.broadcasted_iota(jnp.int32, (1, _KEFF), 1)
    l_iota = jax.lax.broadcasted_iota(jnp.int32, (1, _LANES), 1)

    s_l, y_l, m_l, lc8_l = [], [], [], []
    for bi in range(_NB):
        chunks = []
        for c0 in range(0, _N, _CH):
            xc = xt_ref[bi, :, c0 : c0 + _CH]
            h = jnp.tanh(
                jnp.dot(w1t_ref[...], xc, preferred_element_type=jnp.float32)
                + b1_ref[...]
            )
            es = jnp.dot(w2_ref[...], h, preferred_element_type=jnp.float32)
            es = es + b2_ref[0, 0]
            chunks.append(jax.nn.softplus(es).reshape(_CH // _LANES, _LANES))
        s = jnp.concatenate(chunks, axis=0)

        t = s * (_RSEL / _LAM)
        m = jnp.max(t)
        e = jnp.exp(t - m)
        z = jnp.sum(e)
        c8 = _KSEL / z
        y = e * c8
        y_ref[bi] = y
        s_l.append(s)
        y_l.append(y)
        m_l.append(m)
        lc8_l.append(jnp.log(c8))

    yw_l = list(y_l)
    salrow_l = [jnp.zeros((1, _KEFF), jnp.float32) for _ in range(_NB)]
    posrow_l = [jnp.zeros((1, _KEFF), jnp.float32) for _ in range(_NB)]
    cumrow_l = [jnp.zeros((1, _KEFF), jnp.float32) for _ in range(_NB)]
    offs = [[None] * _KEFF for _ in range(_NB)]
    copies = []
    for k in range(_KEFF):
        for bi in range(_NB):
            yw = yw_l[bi]
            v = jnp.max(yw)
            idxk = jnp.min(jnp.where(yw == v, flat, jnp.int32(_N)))
            sk = bi * _KEFF + k
            idx_smem[sk] = (idxk // _LANES) * _LANES
            offs[bi][k] = idxk - idx_smem[sk]
            cp = pltpu.make_async_copy(
                x_any.at[
                    i * _NB + bi, :,
                    pl.ds(pl.multiple_of(idx_smem[sk], _LANES), _LANES),
                ],
                rows_vmem.at[:, pl.ds(sk * _LANES, _LANES)],
                sems.at[sk],
            )
            cp.start()
            copies.append(cp)

            salk = 0.5 * (jnp.log(v) - lc8_l[bi] + m_l[bi])
            posk = idxk.astype(jnp.float32) * (1.0 / (_N - 1))
            cumk = jnp.sum(jnp.where(flat <= idxk, s_l[bi], 0.0)) * (1.0 / _N)
            km = k16_iota == k
            salrow_l[bi] = jnp.where(km, salk, salrow_l[bi])
            posrow_l[bi] = jnp.where(km, posk, posrow_l[bi])
            cumrow_l[bi] = jnp.where(km, cumk, cumrow_l[bi])
            yw_l[bi] = jnp.where(flat == idxk, -1.0, yw)

    for cp in copies:
        cp.wait()

    for bi in range(_NB):
        cols = []
        for k in range(_KEFF):
            sk = bi * _KEFF + k
            tile = rows_vmem[:, sk * _LANES : (sk + 1) * _LANES]
            sel = jnp.where(l_iota == offs[bi][k], tile, 0.0)
            cols.append(jnp.sum(sel, axis=1, keepdims=True))
        rows = jnp.concatenate(cols, axis=1)

        salrow, posrow, cumrow = salrow_l[bi], posrow_l[bi], cumrow_l[bi]
        n2 = (
            jnp.sum(rows * rows, axis=0, keepdims=True)
            + salrow * salrow + posrow * posrow + cumrow * cumrow
        )
        denom = jnp.sqrt(n2) + 1e-6
        gt = (
            jnp.dot(wlxt_ref[...], rows, preferred_element_type=jnp.float32)
            + wlst_ref[...] * salrow
            + wlpt_ref[...] * posrow
            + wlct_ref[...] * cumrow
        )
        lifted_t = jnp.tanh(gt / denom + blt_ref[...])
        tok = jax.lax.dot_general(
            lifted_t, wp_ref[...],
            dimension_numbers=(((0,), (0,)), ((), ())),
            preferred_element_type=jnp.float32,
        )
        tok_ref[bi] = tok + bp_ref[...]


def kernel(x, W1, b1, W2, b2, W_lift, b_lift, Wp, bp):
    B, N, IN = x.shape
    d_model = Wp.shape[1]

    xt = jnp.transpose(x, (0, 2, 1))

    y_star, tokens = pl.pallas_call(
        _fused_kernel,
        grid=(B // _NB,),
        in_specs=[
            pl.BlockSpec((_NB, IN, N), lambda i: (i, 0, 0)),
            pl.BlockSpec(memory_space=pl.ANY),
            pl.BlockSpec((_HID, IN), lambda i: (0, 0)),
            pl.BlockSpec((_HID, 1), lambda i: (0, 0)),
            pl.BlockSpec((1, _HID), lambda i: (0, 0)),
            pl.BlockSpec((1, 1), lambda i: (0, 0)),
            pl.BlockSpec((16, IN), lambda i: (0, 0)),
            pl.BlockSpec((16, 1), lambda i: (0, 0)),
            pl.BlockSpec((16, 1), lambda i: (0, 0)),
            pl.BlockSpec((16, 1), lambda i: (0, 0)),
            pl.BlockSpec((16, 1), lambda i: (0, 0)),
            pl.BlockSpec((16, d_model), lambda i: (0, 0)),
            pl.BlockSpec((1, d_model), lambda i: (0, 0)),
        ],
        out_specs=[
            pl.BlockSpec((_NB, _ROWS, _LANES), lambda i: (i, 0, 0)),
            pl.BlockSpec((_NB, _KEFF, d_model), lambda i: (i, 0, 0)),
        ],
        out_shape=[
            jax.ShapeDtypeStruct((B, _ROWS, _LANES), jnp.float32),
            jax.ShapeDtypeStruct((B, _KEFF, d_model), jnp.float32),
        ],
        scratch_shapes=[
            pltpu.VMEM((_IN, _NB * _KEFF * _LANES), jnp.float32),
            pltpu.SMEM((_NB * _KEFF,), jnp.int32),
            pltpu.SemaphoreType.DMA((_NB * _KEFF,)),
        ],
    )(
        xt,
        xt,
        W1.T,
        b1.reshape(_HID, 1),
        W2.T,
        b2.reshape(1, 1),
        W_lift[:IN].T,
        W_lift[IN : IN + 1].T,
        W_lift[IN + 1 : IN + 2].T,
        W_lift[IN + 2 : IN + 3].T,
        b_lift.reshape(16, 1),
        Wp,
        bp.reshape(1, d_model),
    )

    return tokens, y_star.reshape(B, N)


# speedup vs baseline: 4.3578x; 1.0012x over previous
import jax
import jax.numpy as jnp
from jax.experimental import pallas as pl
from jax.experimental.pallas import tpu as pltpu

_B, _N, _IN = 16, 32768, 32
_HID = 64
_KSEL = 8
_LAM = 0.5
_RSEL = 1.0
_KEFF = 16
_CH = 8192
_LANES = 128
_ROWS = _N // _LANES
_NB = 4


def _fused_kernel(xt_ref, x_any, w1t_ref, b1_ref, w2_ref, b2_ref,
                  wlxt_ref, wlst_ref, wlpt_ref, wlct_ref, blt_ref,
                  wp_ref, bp_ref,
                  y_ref, tok_ref,
                  rows_vmem, idx_smem, sems):
    i = pl.program_id(0)

    r_iota = jax.lax.broadcasted_iota(jnp.int32, (_ROWS, _LANES), 0)
    c_iota = jax.lax.broadcasted_iota(jnp.int32, (_ROWS, _LANES), 1)
    flat = r_iota * _LANES + c_iota
    k16_iota = jax.lax.broadcasted_iota(jnp.int32, (1, _KEFF), 1)
    l_iota = jax.lax.broadcasted_iota(jnp.int32, (1, _LANES), 1)

    s_l, y_l, m_l, lc8_l = [], [], [], []
    for bi in range(_NB):
        chunks = []
        for c0 in range(0, _N, _CH):
            xc = xt_ref[bi, :, c0 : c0 + _CH]
            h = jnp.tanh(
                jnp.dot(w1t_ref[...], xc, preferred_element_type=jnp.float32)
                + b1_ref[...]
            )
            es = jnp.dot(w2_ref[...], h, preferred_element_type=jnp.float32)
            es = es + b2_ref[0, 0]
            chunks.append(jax.nn.softplus(es).reshape(_CH // _LANES, _LANES))
        s = jnp.concatenate(chunks, axis=0)

        t = s * (_RSEL / _LAM)
        m = jnp.max(t)
        e = jnp.exp(t - m)
        z = jnp.sum(e)
        c8 = _KSEL / z
        y = e * c8
        y_ref[bi] = y
        s_l.append(s)
        y_l.append(y)
        m_l.append(m)
        lc8_l.append(jnp.log(c8))

    yw_l = list(y_l)
    salrow_l = [jnp.zeros((1, _KEFF), jnp.float32) for _ in range(_NB)]
    posrow_l = [jnp.zeros((1, _KEFF), jnp.float32) for _ in range(_NB)]
    cumrow_l = [jnp.zeros((1, _KEFF), jnp.float32) for _ in range(_NB)]
    offs = [[None] * _KEFF for _ in range(_NB)]
    copies = []
    for k in range(_KEFF):
        for bi in range(_NB):
            yw = yw_l[bi]
            v = jnp.max(yw)
            idxk = jnp.min(jnp.where(yw == v, flat, jnp.int32(_N)))
            sk = bi * _KEFF + k
            idx_smem[sk] = (idxk // _LANES) * _LANES
            offs[bi][k] = idxk - idx_smem[sk]
            cp = pltpu.make_async_copy(
                x_any.at[
                    i * _NB + bi, :,
                    pl.ds(pl.multiple_of(idx_smem[sk], _LANES), _LANES),
                ],
                rows_vmem.at[:, pl.ds(sk * _LANES, _LANES)],
                sems.at[sk],
            )
            cp.start()
            copies.append(cp)

            salk = 0.5 * (jnp.log(v) - lc8_l[bi] + m_l[bi])
            posk = idxk.astype(jnp.float32) * (1.0 / (_N - 1))
            cumk = jnp.sum(jnp.where(flat <= idxk, s_l[bi], 0.0)) * (1.0 / _N)
            km = k16_iota == k
            salrow_l[bi] = jnp.where(km, salk, salrow_l[bi])
            posrow_l[bi] = jnp.where(km, posk, posrow_l[bi])
            cumrow_l[bi] = jnp.where(km, cumk, cumrow_l[bi])
            yw_l[bi] = jnp.where(flat == idxk, -1.0, yw)

    for cp in copies:
        cp.wait()

    for bi in range(_NB):
        cols = []
        for k in range(_KEFF):
            sk = bi * _KEFF + k
            tile = rows_vmem[:, sk * _LANES : (sk + 1) * _LANES]
            sel = jnp.where(l_iota == offs[bi][k], tile, 0.0)
            cols.append(jnp.sum(sel, axis=1, keepdims=True))
        rows = jnp.concatenate(cols, axis=1)

        salrow, posrow, cumrow = salrow_l[bi], posrow_l[bi], cumrow_l[bi]
        n2 = (
            jnp.sum(rows * rows, axis=0, keepdims=True)
            + salrow * salrow + posrow * posrow + cumrow * cumrow
        )
        denom = jnp.sqrt(n2) + 1e-6
        gt = (
            jnp.dot(wlxt_ref[...], rows, preferred_element_type=jnp.float32)
            + wlst_ref[...] * salrow
            + wlpt_ref[...] * posrow
            + wlct_ref[...] * cumrow
        )
        lifted_t = jnp.tanh(gt / denom + blt_ref[...])
        tok = jax.lax.dot_general(
            lifted_t, wp_ref[...],
            dimension_numbers=(((0,), (0,)), ((), ())),
            preferred_element_type=jnp.float32,
        )
        tok_ref[bi] = tok + bp_ref[...]


def kernel(x, W1, b1, W2, b2, W_lift, b_lift, Wp, bp):
    B, N, IN = x.shape
    d_model = Wp.shape[1]

    xt = jnp.transpose(x, (0, 2, 1))

    y_star, tokens = pl.pallas_call(
        _fused_kernel,
        grid=(B // _NB,),
        in_specs=[
            pl.BlockSpec((_NB, IN, N), lambda i: (i, 0, 0)),
            pl.BlockSpec(memory_space=pl.ANY),
            pl.BlockSpec((_HID, IN), lambda i: (0, 0)),
            pl.BlockSpec((_HID, 1), lambda i: (0, 0)),
            pl.BlockSpec((1, _HID), lambda i: (0, 0)),
            pl.BlockSpec((1, 1), lambda i: (0, 0)),
            pl.BlockSpec((16, IN), lambda i: (0, 0)),
            pl.BlockSpec((16, 1), lambda i: (0, 0)),
            pl.BlockSpec((16, 1), lambda i: (0, 0)),
            pl.BlockSpec((16, 1), lambda i: (0, 0)),
            pl.BlockSpec((16, 1), lambda i: (0, 0)),
            pl.BlockSpec((16, d_model), lambda i: (0, 0)),
            pl.BlockSpec((1, d_model), lambda i: (0, 0)),
        ],
        out_specs=[
            pl.BlockSpec((_NB, _ROWS, _LANES), lambda i: (i, 0, 0)),
            pl.BlockSpec((_NB, _KEFF, d_model), lambda i: (i, 0, 0)),
        ],
        out_shape=[
            jax.ShapeDtypeStruct((B, _ROWS, _LANES), jnp.float32),
            jax.ShapeDtypeStruct((B, _KEFF, d_model), jnp.float32),
        ],
        scratch_shapes=[
            pltpu.VMEM((_IN, _NB * _KEFF * _LANES), jnp.float32),
            pltpu.SMEM((_NB * _KEFF,), jnp.int32),
            pltpu.SemaphoreType.DMA((_NB * _KEFF,)),
        ],
        compiler_params=pltpu.CompilerParams(
            dimension_semantics=("parallel",),
        ),
    )(
        xt,
        xt,
        W1.T,
        b1.reshape(_HID, 1),
        W2.T,
        b2.reshape(1, 1),
        W_lift[:IN].T,
        W_lift[IN : IN + 1].T,
        W_lift[IN + 1 : IN + 2].T,
        W_lift[IN + 2 : IN + 3].T,
        b_lift.reshape(16, 1),
        Wp,
        bp.reshape(1, d_model),
    )

    return tokens, y_star.reshape(B, N)


# speedup vs baseline: 4.4796x; 1.0280x over previous
import jax
import jax.numpy as jnp
from jax.experimental import pallas as pl
from jax.experimental.pallas import tpu as pltpu

_B, _N, _IN = 16, 32768, 32
_HID = 64
_KSEL = 8
_LAM = 0.5
_RSEL = 1.0
_KEFF = 16
_CH = 8192
_LANES = 128
_ROWS = _N // _LANES
_NB = 4


_TLHS = (((0,), (0,)), ((), ()))


def _fused_kernel(xt_ref, x_any, w1_ref, b1_ref, w2_ref, b2_ref,
                  wl_ref, bl_ref, wp_ref, bp_ref,
                  y_ref, tok_ref,
                  rows_vmem, idx_smem, sems):
    i = pl.program_id(0)
    b1col = b1_ref[...].T
    blcol = bl_ref[...].T

    r_iota = jax.lax.broadcasted_iota(jnp.int32, (_ROWS, _LANES), 0)
    c_iota = jax.lax.broadcasted_iota(jnp.int32, (_ROWS, _LANES), 1)
    flat = r_iota * _LANES + c_iota
    k16_iota = jax.lax.broadcasted_iota(jnp.int32, (1, _KEFF), 1)
    l_iota = jax.lax.broadcasted_iota(jnp.int32, (1, _LANES), 1)

    s_l, y_l, m_l, lc8_l = [], [], [], []
    for bi in range(_NB):
        chunks = []
        for c0 in range(0, _N, _CH):
            xc = xt_ref[bi, :, c0 : c0 + _CH]
            h = jnp.tanh(
                jax.lax.dot_general(
                    w1_ref[...], xc, dimension_numbers=_TLHS,
                    preferred_element_type=jnp.float32,
                )
                + b1col
            )
            es = jax.lax.dot_general(
                w2_ref[...], h, dimension_numbers=_TLHS,
                preferred_element_type=jnp.float32,
            )
            es = es + b2_ref[0, 0]
            chunks.append(jax.nn.softplus(es).reshape(_CH // _LANES, _LANES))
        s = jnp.concatenate(chunks, axis=0)

        t = s * (_RSEL / _LAM)
        m = jnp.max(t)
        e = jnp.exp(t - m)
        z = jnp.sum(e)
        c8 = _KSEL / z
        y = e * c8
        y_ref[bi] = y
        s_l.append(s)
        y_l.append(y)
        m_l.append(m)
        lc8_l.append(jnp.log(c8))

    yw_l = list(y_l)
    salrow_l = [jnp.zeros((1, _KEFF), jnp.float32) for _ in range(_NB)]
    posrow_l = [jnp.zeros((1, _KEFF), jnp.float32) for _ in range(_NB)]
    cumrow_l = [jnp.zeros((1, _KEFF), jnp.float32) for _ in range(_NB)]
    offs = [[None] * _KEFF for _ in range(_NB)]
    copies = []
    for k in range(_KEFF):
        for bi in range(_NB):
            yw = yw_l[bi]
            v = jnp.max(yw)
            idxk = jnp.min(jnp.where(yw == v, flat, jnp.int32(_N)))
            sk = bi * _KEFF + k
            idx_smem[sk] = (idxk // _LANES) * _LANES
            offs[bi][k] = idxk - idx_smem[sk]
            cp = pltpu.make_async_copy(
                x_any.at[
                    i * _NB + bi, :,
                    pl.ds(pl.multiple_of(idx_smem[sk], _LANES), _LANES),
                ],
                rows_vmem.at[:, pl.ds(sk * _LANES, _LANES)],
                sems.at[sk],
            )
            cp.start()
            copies.append(cp)

            salk = 0.5 * (jnp.log(v) - lc8_l[bi] + m_l[bi])
            posk = idxk.astype(jnp.float32) * (1.0 / (_N - 1))
            cumk = jnp.sum(jnp.where(flat <= idxk, s_l[bi], 0.0)) * (1.0 / _N)
            km = k16_iota == k
            salrow_l[bi] = jnp.where(km, salk, salrow_l[bi])
            posrow_l[bi] = jnp.where(km, posk, posrow_l[bi])
            cumrow_l[bi] = jnp.where(km, cumk, cumrow_l[bi])
            yw_l[bi] = jnp.where(flat == idxk, -1.0, yw)

    for cp in copies:
        cp.wait()

    for bi in range(_NB):
        cols = []
        for k in range(_KEFF):
            sk = bi * _KEFF + k
            tile = rows_vmem[:, sk * _LANES : (sk + 1) * _LANES]
            sel = jnp.where(l_iota == offs[bi][k], tile, 0.0)
            cols.append(jnp.sum(sel, axis=1, keepdims=True))
        rows = jnp.concatenate(cols, axis=1)

        aug = jnp.concatenate(
            [rows, salrow_l[bi], posrow_l[bi], cumrow_l[bi]], axis=0
        )
        n2 = jnp.sum(aug * aug, axis=0, keepdims=True)
        denom = jnp.sqrt(n2) + 1e-6
        gt = jax.lax.dot_general(
            wl_ref[...], aug, dimension_numbers=_TLHS,
            preferred_element_type=jnp.float32,
        )
        lifted_t = jnp.tanh(gt / denom + blcol)
        tok = jax.lax.dot_general(
            lifted_t, wp_ref[...],
            dimension_numbers=_TLHS,
            preferred_element_type=jnp.float32,
        )
        tok_ref[bi] = tok + bp_ref[...]


def kernel(x, W1, b1, W2, b2, W_lift, b_lift, Wp, bp):
    B, N, IN = x.shape
    d_model = Wp.shape[1]

    xt = jnp.transpose(x, (0, 2, 1))

    y_star, tokens = pl.pallas_call(
        _fused_kernel,
        grid=(B // _NB,),
        in_specs=[
            pl.BlockSpec((_NB, IN, N), lambda i: (i, 0, 0)),
            pl.BlockSpec(memory_space=pl.ANY),
            pl.BlockSpec((IN, _HID), lambda i: (0, 0)),
            pl.BlockSpec((1, _HID), lambda i: (0, 0)),
            pl.BlockSpec((_HID, 1), lambda i: (0, 0)),
            pl.BlockSpec((1, 1), lambda i: (0, 0)),
            pl.BlockSpec((35, 16), lambda i: (0, 0)),
            pl.BlockSpec((1, 16), lambda i: (0, 0)),
            pl.BlockSpec((16, d_model), lambda i: (0, 0)),
            pl.BlockSpec((1, d_model), lambda i: (0, 0)),
        ],
        out_specs=[
            pl.BlockSpec((_NB, _ROWS, _LANES), lambda i: (i, 0, 0)),
            pl.BlockSpec((_NB, _KEFF, d_model), lambda i: (i, 0, 0)),
        ],
        out_shape=[
            jax.ShapeDtypeStruct((B, _ROWS, _LANES), jnp.float32),
            jax.ShapeDtypeStruct((B, _KEFF, d_model), jnp.float32),
        ],
        scratch_shapes=[
            pltpu.VMEM((_IN, _NB * _KEFF * _LANES), jnp.float32),
            pltpu.SMEM((_NB * _KEFF,), jnp.int32),
            pltpu.SemaphoreType.DMA((_NB * _KEFF,)),
        ],
        compiler_params=pltpu.CompilerParams(
            dimension_semantics=("parallel",),
        ),
    )(
        xt,
        xt,
        W1,
        b1.reshape(1, _HID),
        W2,
        b2.reshape(1, 1),
        W_lift,
        b_lift.reshape(1, 16),
        Wp,
        bp.reshape(1, d_model),
    )

    return tokens, y_star.reshape(B, N)


# speedup vs baseline: 4.5103x; 1.0069x over previous
import jax
import jax.numpy as jnp
from jax.experimental import pallas as pl
from jax.experimental.pallas import tpu as pltpu

_B, _N, _IN = 16, 32768, 32
_HID = 64
_KSEL = 8
_LAM = 0.5
_RSEL = 1.0
_KEFF = 16
_CH = 16384
_LANES = 128
_ROWS = _N // _LANES
_NB = 4


_TLHS = (((0,), (0,)), ((), ()))


def _fused_kernel(xt_ref, x_any, w1_ref, b1_ref, w2_ref, b2_ref,
                  wl_ref, bl_ref, wp_ref, bp_ref,
                  y_ref, tok_ref,
                  rows_vmem, idx_smem, sems):
    i = pl.program_id(0)
    b1col = b1_ref[...].T
    blcol = bl_ref[...].T

    r_iota = jax.lax.broadcasted_iota(jnp.int32, (_ROWS, _LANES), 0)
    c_iota = jax.lax.broadcasted_iota(jnp.int32, (_ROWS, _LANES), 1)
    flat = r_iota * _LANES + c_iota
    k16_iota = jax.lax.broadcasted_iota(jnp.int32, (1, _KEFF), 1)
    l_iota = jax.lax.broadcasted_iota(jnp.int32, (1, _LANES), 1)

    s_l, y_l, m_l, lc8_l = [], [], [], []
    for bi in range(_NB):
        chunks = []
        for c0 in range(0, _N, _CH):
            xc = xt_ref[bi, :, c0 : c0 + _CH]
            h = jnp.tanh(
                jax.lax.dot_general(
                    w1_ref[...], xc, dimension_numbers=_TLHS,
                    preferred_element_type=jnp.float32,
                )
                + b1col
            )
            es = jax.lax.dot_general(
                w2_ref[...], h, dimension_numbers=_TLHS,
                preferred_element_type=jnp.float32,
            )
            es = es + b2_ref[0, 0]
            chunks.append(jax.nn.softplus(es).reshape(_CH // _LANES, _LANES))
        s = jnp.concatenate(chunks, axis=0)

        t = s * (_RSEL / _LAM)
        m = jnp.max(t)
        e = jnp.exp(t - m)
        z = jnp.sum(e)
        c8 = _KSEL / z
        y = e * c8
        y_ref[bi] = y
        s_l.append(s)
        y_l.append(y)
        m_l.append(m)
        lc8_l.append(jnp.log(c8))

    yw_l = list(y_l)
    salrow_l = [jnp.zeros((1, _KEFF), jnp.float32) for _ in range(_NB)]
    posrow_l = [jnp.zeros((1, _KEFF), jnp.float32) for _ in range(_NB)]
    cumrow_l = [jnp.zeros((1, _KEFF), jnp.float32) for _ in range(_NB)]
    offs = [[None] * _KEFF for _ in range(_NB)]
    copies = []
    for k in range(_KEFF):
        for bi in range(_NB):
            yw = yw_l[bi]
            v = jnp.max(yw)
            idxk = jnp.min(jnp.where(yw == v, flat, jnp.int32(_N)))
            sk = bi * _KEFF + k
            idx_smem[sk] = (idxk // _LANES) * _LANES
            offs[bi][k] = idxk - idx_smem[sk]
            cp = pltpu.make_async_copy(
                x_any.at[
                    i * _NB + bi, :,
                    pl.ds(pl.multiple_of(idx_smem[sk], _LANES), _LANES),
                ],
                rows_vmem.at[:, pl.ds(sk * _LANES, _LANES)],
                sems.at[sk],
            )
            cp.start()
            copies.append(cp)

            salk = 0.5 * (jnp.log(v) - lc8_l[bi] + m_l[bi])
            posk = idxk.astype(jnp.float32) * (1.0 / (_N - 1))
            cumk = jnp.sum(jnp.where(flat <= idxk, s_l[bi], 0.0)) * (1.0 / _N)
            km = k16_iota == k
            salrow_l[bi] = jnp.where(km, salk, salrow_l[bi])
            posrow_l[bi] = jnp.where(km, posk, posrow_l[bi])
            cumrow_l[bi] = jnp.where(km, cumk, cumrow_l[bi])
            yw_l[bi] = jnp.where(flat == idxk, -1.0, yw)

    for cp in copies:
        cp.wait()

    for bi in range(_NB):
        cols = []
        for k in range(_KEFF):
            sk = bi * _KEFF + k
            tile = rows_vmem[:, sk * _LANES : (sk + 1) * _LANES]
            sel = jnp.where(l_iota == offs[bi][k], tile, 0.0)
            cols.append(jnp.sum(sel, axis=1, keepdims=True))
        rows = jnp.concatenate(cols, axis=1)

        aug = jnp.concatenate(
            [rows, salrow_l[bi], posrow_l[bi], cumrow_l[bi]], axis=0
        )
        n2 = jnp.sum(aug * aug, axis=0, keepdims=True)
        denom = jnp.sqrt(n2) + 1e-6
        gt = jax.lax.dot_general(
            wl_ref[...], aug, dimension_numbers=_TLHS,
            preferred_element_type=jnp.float32,
        )
        lifted_t = jnp.tanh(gt / denom + blcol)
        tok = jax.lax.dot_general(
            lifted_t, wp_ref[...],
            dimension_numbers=_TLHS,
            preferred_element_type=jnp.float32,
        )
        tok_ref[bi] = tok + bp_ref[...]


def kernel(x, W1, b1, W2, b2, W_lift, b_lift, Wp, bp):
    B, N, IN = x.shape
    d_model = Wp.shape[1]

    xt = jnp.transpose(x, (0, 2, 1))

    y_star, tokens = pl.pallas_call(
        _fused_kernel,
        grid=(B // _NB,),
        in_specs=[
            pl.BlockSpec((_NB, IN, N), lambda i: (i, 0, 0)),
            pl.BlockSpec(memory_space=pl.ANY),
            pl.BlockSpec((IN, _HID), lambda i: (0, 0)),
            pl.BlockSpec((1, _HID), lambda i: (0, 0)),
            pl.BlockSpec((_HID, 1), lambda i: (0, 0)),
            pl.BlockSpec((1, 1), lambda i: (0, 0)),
            pl.BlockSpec((35, 16), lambda i: (0, 0)),
            pl.BlockSpec((1, 16), lambda i: (0, 0)),
            pl.BlockSpec((16, d_model), lambda i: (0, 0)),
            pl.BlockSpec((1, d_model), lambda i: (0, 0)),
        ],
        out_specs=[
            pl.BlockSpec((_NB, _ROWS, _LANES), lambda i: (i, 0, 0)),
            pl.BlockSpec((_NB, _KEFF, d_model), lambda i: (i, 0, 0)),
        ],
        out_shape=[
            jax.ShapeDtypeStruct((B, _ROWS, _LANES), jnp.float32),
            jax.ShapeDtypeStruct((B, _KEFF, d_model), jnp.float32),
        ],
        scratch_shapes=[
            pltpu.VMEM((_IN, _NB * _KEFF * _LANES), jnp.float32),
            pltpu.SMEM((_NB * _KEFF,), jnp.int32),
            pltpu.SemaphoreType.DMA((_NB * _KEFF,)),
        ],
        compiler_params=pltpu.CompilerParams(
            dimension_semantics=("parallel",),
        ),
    )(
        xt,
        xt,
        W1,
        b1.reshape(1, _HID),
        W2,
        b2.reshape(1, 1),
        W_lift,
        b_lift.reshape(1, 16),
        Wp,
        bp.reshape(1, d_model),
    )

    return tokens, y_star.reshape(B, N)
